# Initial kernel scaffold; baseline (speedup 1.0000x reference)
import jax
import jax.numpy as jnp
from jax.experimental import pallas as pl


def kernel(x, edge_index, W1, b1, W2, b2, Wl1, bl1, Wl2, bl2):
    raise NotImplementedError("write your pallas kernel here")


# speedup vs baseline: 11.1392x; 11.1392x over previous
import functools

import jax
import jax.numpy as jnp
from jax import lax
from jax.experimental import pallas as pl
from jax.experimental.pallas import tpu as pltpu
from jax.experimental.pallas import tpu_sc as plsc

N = 10000
E = 320000
D = 128
NC = 2
NS = 16
NW = NC * NS
CHUNK = 64
IB = 16
C = IB * -(-E // (NW * CHUNK * IB))
NBLK = C // IB
EPT = C * CHUNK
E_PAD = NW * EPT
RPT = 640
NIB = RPT // CHUNK
N_ACC = RPT * NS


def _sc_pass_body(with_cnt, *refs):
    if with_cnt:
        (table, src3, dst3, zfull, agg_out, cnt_out,
         src_v, dst_v, buf0, buf1, cnt_v, iidx_v, sem0, sem1,
         agg_acc) = refs
    else:
        (table, src3, dst3, zfull, agg_out,
         src_v, dst_v, buf0, buf1, iidx_v, sem0, sem1, agg_acc) = refs
    cid = lax.axis_index("c")
    sid = lax.axis_index("s")
    wid = cid * NS + sid

    def irow(c):
        return ((c // IB) % 2) * IB + (c % IB)

    def load_iblock(b):
        pltpu.sync_copy(src3.at[wid, pl.ds(b * IB, IB)],
                        src_v.at[pl.ds((b % 2) * IB, IB)])
        pltpu.sync_copy(dst3.at[wid, pl.ds(b * IB, IB)],
                        dst_v.at[pl.ds((b % 2) * IB, IB)])

    iota = lax.iota(jnp.int32, 16)
    for k in range(NIB):
        for j in range(CHUNK // 16):
            iidx_v[k, pl.ds(j * 16, 16)] = sid * RPT + k * CHUNK + j * 16 + iota

    load_iblock(0)

    pltpu.sync_copy(zfull, buf0)
    for k in range(NIB):
        pltpu.sync_copy(buf0, agg_acc.at[iidx_v.at[k]])
    plsc.subcore_barrier()
    if with_cnt:
        def zc(i, carry):
            cnt_v[pl.ds(i * 16, 16)] = jnp.zeros((16,), jnp.float32)
            return carry
        lax.fori_loop(0, N_ACC // 16, zc, 0)

    pltpu.async_copy(table.at[src_v.at[0]], buf0, sem0)
    pltpu.async_copy(table.at[src_v.at[1]], buf1, sem1)

    def step(i, carry):
        c0 = 2 * i

        @pl.when(c0 % IB == 0)
        def _():
            nb = c0 // IB + 1

            @pl.when(nb < NBLK)
            def _():
                load_iblock(nb)

        for c, buf, sem in ((c0, buf0, sem0), (c0 + 1, buf1, sem1)):
            pltpu.make_async_copy(table.at[src_v.at[irow(c)]], buf,
                                  sem).wait()
            pltpu.sync_copy(buf, agg_acc.at[dst_v.at[irow(c)]], add=True)
            if with_cnt:
                fones = jnp.ones((16,), jnp.float32)
                for j in range(CHUNK // 16):
                    idx16 = dst_v[irow(c), pl.ds(j * 16, 16)]
                    plsc.addupdate_scatter(cnt_v, [idx16], fones)

            @pl.when(i < C // 2 - 1)
            def _():
                pltpu.async_copy(table.at[src_v.at[irow(c + 2)]], buf, sem)
        return carry

    lax.fori_loop(0, C // 2, step, 0)
    plsc.subcore_barrier()

    for k in range(NIB):
        base = sid * RPT + k * CHUNK
        pltpu.async_copy(agg_acc.at[iidx_v.at[k]], buf0, sem0).wait()
        pltpu.sync_copy(buf0, agg_out.at[cid, pl.ds(base, CHUNK)])
    if with_cnt:
        pltpu.sync_copy(cnt_v, cnt_out.at[wid])


@functools.cache
def _sc_pass(with_cnt):
    out_type = [jax.ShapeDtypeStruct((NC, N_ACC, D), jnp.float32)]
    scratch = [
        pltpu.VMEM((2 * IB, CHUNK), jnp.int32),
        pltpu.VMEM((2 * IB, CHUNK), jnp.int32),
        pltpu.VMEM((CHUNK, D), jnp.float32),
        pltpu.VMEM((CHUNK, D), jnp.float32),
    ]
    if with_cnt:
        out_type.append(jax.ShapeDtypeStruct((NW, N_ACC), jnp.float32))
        scratch.append(pltpu.VMEM((N_ACC,), jnp.float32))
    scratch += [pltpu.VMEM((NIB, CHUNK), jnp.int32),
                pltpu.SemaphoreType.DMA, pltpu.SemaphoreType.DMA,
                pltpu.VMEM_SHARED((N_ACC, D), jnp.float32)]
    return pl.kernel(
        functools.partial(_sc_pass_body, with_cnt),
        out_type=out_type,
        mesh=plsc.VectorSubcoreMesh(core_axis_name="c", subcore_axis_name="s",
                                    num_cores=NC, num_subcores=NS),
        scratch_types=scratch,
        compiler_params=pltpu.CompilerParams(needs_layout_passes=False),
    )


BLK = 2000


def _inv_cnt_body(cntp, out):
    ones = jnp.ones((NW, 1), jnp.float32)
    cnt = lax.dot_general(cntp[...], ones, (((0,), (0,)), ((), ())),
                          preferred_element_type=jnp.float32) + 1.0
    out[...] = 1.0 / cnt


def _inv_cnt_call(cntp):
    return pl.pallas_call(
        _inv_cnt_body,
        out_shape=jax.ShapeDtypeStruct((N_ACC, 1), jnp.float32),
    )(cntp)


def _l1_body(aggp, invc, x, w1a, w1b, b1, out):
    agg = aggp[0] + aggp[1] + x[...]
    mean = agg * invc[...]
    h = (jnp.dot(mean, w1a[...], preferred_element_type=jnp.float32)
         + jnp.dot(x[...], w1b[...], preferred_element_type=jnp.float32)
         + b1[...])
    out[...] = jnp.maximum(h, 0.0)


def _l2_body(aggp, invc, h1, w2a, w2b, b2, wl1, bl1, wl2, bl2, out):
    agg = aggp[0] + aggp[1] + h1[...]
    mean = agg * invc[...]
    h = (jnp.dot(mean, w2a[...], preferred_element_type=jnp.float32)
         + jnp.dot(h1[...], w2b[...], preferred_element_type=jnp.float32)
         + b2[...])
    h = jnp.maximum(h, 0.0)
    s = jnp.dot(h, wl1[...], preferred_element_type=jnp.float32) + bl1[...]
    s = jnp.maximum(s, 0.0)
    out[...] = jnp.dot(s, wl2[...], preferred_element_type=jnp.float32) + bl2[...]


def _wspec(shape):
    return pl.BlockSpec(shape, lambda i: (0,) * len(shape))


def _l1_call(aggp, invc, x, w1a, w1b, b1):
    return pl.pallas_call(
        _l1_body,
        grid=(N // BLK,),
        in_specs=[
            pl.BlockSpec((NC, BLK, D), lambda i: (0, i, 0)),
            pl.BlockSpec((BLK, 1), lambda i: (i, 0)),
            pl.BlockSpec((BLK, D), lambda i: (i, 0)),
            _wspec((D, D)), _wspec((D, D)), _wspec((1, D)),
        ],
        out_specs=pl.BlockSpec((BLK, D), lambda i: (i, 0)),
        out_shape=jax.ShapeDtypeStruct((N, D), jnp.float32),
    )(aggp, invc, x, w1a, w1b, b1)


def _l2_call(aggp, invc, h1, w2a, w2b, b2, wl1, bl1, wl2, bl2):
    n_cls = wl2.shape[1]
    return pl.pallas_call(
        _l2_body,
        grid=(N // BLK,),
        in_specs=[
            pl.BlockSpec((NC, BLK, D), lambda i: (0, i, 0)),
            pl.BlockSpec((BLK, 1), lambda i: (i, 0)),
            pl.BlockSpec((BLK, D), lambda i: (i, 0)),
            _wspec((D, D)), _wspec((D, D)), _wspec((1, D)),
            _wspec((D, D)), _wspec((1, D)),
            _wspec((D, n_cls)), _wspec((1, n_cls)),
        ],
        out_specs=pl.BlockSpec((BLK, n_cls), lambda i: (i, 0)),
        out_shape=jax.ShapeDtypeStruct((N, n_cls), jnp.float32),
    )(aggp, invc, h1, w2a, w2b, b2, wl1, bl1, wl2, bl2)


def kernel(x, edge_index, W1, b1, W2, b2, Wl1, bl1, Wl2, bl2):
    src = edge_index[0].astype(jnp.int32)
    dst = edge_index[1].astype(jnp.int32)
    pad = E_PAD - E
    pad_src = jnp.arange(pad, dtype=jnp.int32) % N
    pad_dst = N + (jnp.arange(pad, dtype=jnp.int32) % NS)
    src3 = jnp.concatenate([src, pad_src]).reshape(NW, C, CHUNK)
    dst3 = jnp.concatenate([dst, pad_dst]).reshape(NW, C, CHUNK)
    zfull = jnp.zeros((CHUNK, D), jnp.float32)

    aggp, cntp = _sc_pass(True)(x, src3, dst3, zfull)
    invc = _inv_cnt_call(cntp)
    h1 = _l1_call(aggp, invc, x, W1[:D], W1[D:], b1.reshape(1, D))
    (aggp2,) = _sc_pass(False)(h1, src3, dst3, zfull)
    return _l2_call(aggp2, invc, h1, W2[:D], W2[D:], b2.reshape(1, D),
                    Wl1, bl1.reshape(1, D), Wl2, bl2.reshape(1, 2))
